# baseline (speedup 1.0000x reference)
import functools

import jax
import jax.numpy as jnp
from jax import lax
from jax.experimental import pallas as pl
from jax.experimental.pallas import tpu as pltpu
from jax.experimental.pallas import tpu_sc as plsc

N = 50000
E = 800000
F = 32
EPS = 1e-5
SLOPE = 0.1

NCORE = 2
NSUB = 16
CH = 256
SB = 128
NSC = CH // SB
NBUF = 3
EP = 811008
NP = 50048
ROWS_PER_SUB = NP // NSUB


def _sc_spmm2(table, edges_t, edges_s, split_cols):
  nchunks = (EP // NSUB) // CH if split_cols else (EP // (NCORE * NSUB)) // CH
  ntrip = nchunks // NBUF
  mesh = plsc.VectorSubcoreMesh(core_axis_name="c", subcore_axis_name="s")
  zeros = jnp.zeros((NP, F), jnp.float32)

  @functools.partial(
      pl.kernel,
      out_type=jax.ShapeDtypeStruct((2 * NCORE, NP, F), jnp.float32),
      mesh=mesh,
      compiler_params=pltpu.CompilerParams(use_tc_tiling_on_sc=False),
      scratch_types=[
          pltpu.VMEM((NBUF, CH), jnp.int32),
          pltpu.VMEM((NBUF, NSC, SB), jnp.int32),
          pltpu.VMEM((NBUF, CH), jnp.float32),
          pltpu.VMEM((NBUF, CH, F), jnp.float32),
          pltpu.VMEM_SHARED((NP, F), jnp.float32),
          pltpu.SemaphoreType.DMA((NBUF,)),
          pltpu.SemaphoreType.DMA((NBUF,)),
          pltpu.SemaphoreType.DMA((NBUF,)),
      ],
  )
  def spmm(table_h, src_t, dstr_t, w_t, src_s, dstr_s, w_s, zeros_h, out_h,
           srcv, dstv, wv, rows, acc, gsem, ssem, lsem):
    c = lax.axis_index("c")
    s = lax.axis_index("s")
    r0 = pl.multiple_of(s * ROWS_PER_SUB, 8)

    if split_cols:
      base = s * (EP // NSUB)
    else:
      base = (c * NSUB + s) * (EP // (NCORE * NSUB))

    def run_stream(p, src_h, dstr_h, w_h):
      pltpu.sync_copy(zeros_h.at[pl.ds(r0, ROWS_PER_SUB)],
                      acc.at[pl.ds(r0, ROWS_PER_SUB)])
      plsc.subcore_barrier()

      def load_linear(k, b):
        e0 = pl.multiple_of(base + k * CH, CH)
        d1 = pltpu.async_copy(
            src_h.at[pl.ds(pl.multiple_of(c * EP + e0, CH), CH)],
            srcv.at[b], lsem.at[b])
        d2 = pltpu.async_copy(w_h.at[pl.ds(e0, CH)], wv.at[b], lsem.at[b])
        d3 = pltpu.async_copy(
            dstr_h.at[pl.ds(pl.multiple_of(e0 // SB, NSC), NSC)],
            dstv.at[b], lsem.at[b])
        return d1, d2, d3

      def issue_gather(b):
        pltpu.async_copy(table_h.at[srcv.at[b]], rows.at[b], gsem.at[b])

      def wait_gather(b):
        pltpu.make_async_copy(zeros_h.at[pl.ds(0, CH)], rows.at[b],
                              gsem.at[b]).wait()

      def multiply(b):
        def group_body(g, _):
          w16 = wv[b, pl.ds(g * 16, 16)]
          for u in range(16):
            e = g * 16 + u
            ws = w16[u]
            for h in range(0, F, 16):
              rows[b, e, pl.ds(h, 16)] = rows[b, e, pl.ds(h, 16)] * ws
          return 0
        lax.fori_loop(0, CH // 16, group_body, 0)

      def issue_scatters(b):
        for j in range(NSC):
          pltpu.async_copy(rows.at[b, pl.ds(j * SB, SB)],
                           acc.at[dstv.at[b, j]], ssem.at[b], add=True)

      def drain_scatters(b):
        for j in range(NSC):
          pltpu.make_async_copy(zeros_h.at[pl.ds(0, SB)],
                                rows.at[b, pl.ds(j * SB, SB)],
                                ssem.at[b]).wait()

      for b in range(NBUF - 1):
        for d in load_linear(b, b):
          d.wait()
        issue_gather(b)

      def trip_body(t, _):
        for b in range(NBUF):
          k = t * NBUF + b
          bp = (b + NBUF - 1) % NBUF
          wait_gather(b)
          multiply(b)
          issue_scatters(b)

          def refill(j):
            drain_scatters(bp)
            d1, d2, d3 = load_linear(j, bp)
            d1.wait(); d2.wait(); d3.wait()
            issue_gather(bp)

          if b == 0:
            @pl.when(t == 0)
            def _():
              d1, d2, d3 = load_linear(NBUF - 1, bp)
              d1.wait(); d2.wait(); d3.wait()
              issue_gather(bp)

            @pl.when(t > 0)
            def _():
              refill(k + NBUF - 1)
          else:
            @pl.when(t < ntrip - 1)
            def _():
              refill(k + NBUF - 1)

            @pl.when(t == ntrip - 1)
            def _():
              drain_scatters(bp)
        return 0

      lax.fori_loop(0, ntrip, trip_body, 0)
      drain_scatters((nchunks - 1) % NBUF)

      plsc.subcore_barrier()
      pltpu.sync_copy(acc.at[pl.ds(r0, ROWS_PER_SUB)],
                      out_h.at[NCORE * p + c, pl.ds(r0, ROWS_PER_SUB)])

    run_stream(0, src_t, dstr_t, w_t)
    run_stream(1, src_s, dstr_s, w_s)

  out = spmm(table, edges_t[0], edges_t[1], edges_t[2],
             edges_s[0], edges_s[1], edges_s[2], zeros)
  return out.reshape(2, NCORE, NP, F)


BN = 2000


def _tc_mm_body(x_ref, s4_ref, a_ref, b4_ref, bias_ref, y_ref, s1_ref, sq_ref):
  y = jnp.dot(x_ref[0], a_ref[0], preferred_element_type=jnp.float32)
  y -= jnp.dot(s4_ref[0, 0], b4_ref[0, 0], preferred_element_type=jnp.float32)
  y -= jnp.dot(s4_ref[0, 1], b4_ref[0, 1], preferred_element_type=jnp.float32)
  y += bias_ref[0]
  y_ref[0] = y
  part = jnp.sum(y.reshape(BN // 8, 8, F), axis=0)
  psq = jnp.sum((y * y).reshape(BN // 8, 8, F), axis=0)

  @pl.when(pl.program_id(1) == 0)
  def _():
    s1_ref[0] = part
    sq_ref[0] = psq

  @pl.when(pl.program_id(1) != 0)
  def _():
    s1_ref[0] += part
    sq_ref[0] += psq


def _tc_matmul2(x2, s4, a2, b4, bias2):
  d = x2.shape[2]
  return pl.pallas_call(
      _tc_mm_body,
      grid=(2, N // BN),
      in_specs=[
          pl.BlockSpec((1, BN, d), lambda p, i: (p, i, 0)),
          pl.BlockSpec((1, 2, BN, F), lambda p, i: (p, 0, i, 0)),
          pl.BlockSpec((1, d, F), lambda p, i: (p, 0, 0)),
          pl.BlockSpec((1, 2, F, F), lambda p, i: (p, 0, 0, 0)),
          pl.BlockSpec((1, 1, F), lambda p, i: (p, 0, 0)),
      ],
      out_specs=[
          pl.BlockSpec((1, BN, F), lambda p, i: (p, i, 0)),
          pl.BlockSpec((1, 8, F), lambda p, i: (p, 0, 0)),
          pl.BlockSpec((1, 8, F), lambda p, i: (p, 0, 0)),
      ],
      out_shape=[
          jax.ShapeDtypeStruct((2, N, F), jnp.float32),
          jax.ShapeDtypeStruct((2, 8, F), jnp.float32),
          jax.ShapeDtypeStruct((2, 8, F), jnp.float32),
      ],
  )(x2, s4, a2, b4, bias2)


def _tc_norm_body(y_ref, s1_ref, sq_ref, g_ref, b_ref, o_ref):
  s1 = jnp.sum(s1_ref[0], axis=0, keepdims=True)
  sq = jnp.sum(sq_ref[0], axis=0, keepdims=True)
  mean = s1 / N
  var = sq / N - mean * mean
  scale = g_ref[0] * lax.rsqrt(var + EPS)
  shift = b_ref[0] - mean * scale
  o = y_ref[0] * scale + shift
  o_ref[0] = jnp.where(o >= 0, o, SLOPE * o)


def _tc_norm2(y2, s1, sq, gamma2, beta2):
  return pl.pallas_call(
      _tc_norm_body,
      grid=(2, N // BN),
      in_specs=[
          pl.BlockSpec((1, BN, F), lambda p, i: (p, i, 0)),
          pl.BlockSpec((1, 8, F), lambda p, i: (p, 0, 0)),
          pl.BlockSpec((1, 8, F), lambda p, i: (p, 0, 0)),
          pl.BlockSpec((1, 1, F), lambda p, i: (p, 0, 0)),
          pl.BlockSpec((1, 1, F), lambda p, i: (p, 0, 0)),
      ],
      out_specs=pl.BlockSpec((1, BN, F), lambda p, i: (p, i, 0)),
      out_shape=jax.ShapeDtypeStruct((2, N, F), jnp.float32),
  )(y2, s1, sq, gamma2, beta2)


def _combine_weights(Ws):
  ks = jnp.arange(Ws.shape[0], dtype=jnp.float32)
  return jnp.sum(Ws, axis=0), jnp.einsum("k,kij->ij", ks, Ws)


def _prep_edges(ei, w):
  pad = EP - E
  src = jnp.concatenate([ei[0].astype(jnp.int32), jnp.zeros((pad,), jnp.int32)])
  dst = jnp.concatenate([ei[1].astype(jnp.int32), jnp.zeros((pad,), jnp.int32)])
  w_p = jnp.concatenate([w, jnp.zeros((pad,), jnp.float32)])
  return src, dst.reshape(EP // SB, SB), w_p


def _conv_block2(x2, table, edges_t, edges_s, Wt, Ws, bt, bs, gt, gs, bet, bes,
                 split_cols):
  a_t, b_t = _combine_weights(Wt)
  a_s, b_s = _combine_weights(Ws)
  if split_cols:
    b4 = jnp.stack([jnp.stack([b_t[:F], b_t[F:]]),
                    jnp.stack([b_s[:F], b_s[F:]])])
  else:
    b4 = jnp.stack([jnp.stack([b_t, b_t]), jnp.stack([b_s, b_s])])
  a2 = jnp.stack([a_t, a_s])
  bias2 = jnp.stack([bt.reshape(1, F), bs.reshape(1, F)])
  g2 = jnp.stack([gt.reshape(1, F), gs.reshape(1, F)])
  be2 = jnp.stack([bet.reshape(1, F), bes.reshape(1, F)])
  s4 = _sc_spmm2(table, edges_t, edges_s, split_cols)
  y2, s1, sq = _tc_matmul2(x2, s4, a2, b4, bias2)
  return _tc_norm2(y2, s1, sq, g2, be2)


def kernel(x_t0, edge_weight_t, x_s0, edge_weight_s, Wt0, Wt1, bias_t, gamma_t,
           beta_t, Ws0, Ws1, bias_s, gamma_s, beta_s, edge_index_t,
           edge_index_s):
  src_t, dstr_t, wp_t = _prep_edges(edge_index_t, edge_weight_t)
  src_s, dstr_s, wp_s = _prep_edges(edge_index_s, edge_weight_s)

  x2 = jnp.stack([x_t0, x_s0])
  tab0 = jnp.concatenate([x_t0[:, :F], x_t0[:, F:],
                          x_s0[:, :F], x_s0[:, F:]], axis=0)
  e_t0 = (jnp.concatenate([src_t, src_t + N]), dstr_t, wp_t)
  e_s0 = (jnp.concatenate([src_s + 2 * N, src_s + 3 * N]), dstr_s, wp_s)
  h2 = _conv_block2(x2, tab0, e_t0, e_s0, Wt0, Ws0, bias_t[0], bias_s[0],
                    gamma_t[0], gamma_s[0], beta_t[0], beta_s[0],
                    split_cols=True)

  tab1 = h2.reshape(2 * N, F)
  e_t1 = (jnp.concatenate([src_t, src_t]), dstr_t, wp_t)
  e_s1 = (jnp.concatenate([src_s + N, src_s + N]), dstr_s, wp_s)
  o2 = _conv_block2(h2, tab1, e_t1, e_s1, Wt1, Ws1, bias_t[1], bias_s[1],
                    gamma_t[1], gamma_s[1], beta_t[1], beta_s[1],
                    split_cols=False)
  return (o2[0], o2[1])


# speedup vs baseline: 1.1023x; 1.1023x over previous
import functools

import jax
import jax.numpy as jnp
from jax import lax
from jax.experimental import pallas as pl
from jax.experimental.pallas import tpu as pltpu
from jax.experimental.pallas import tpu_sc as plsc

N = 50000
E = 800000
F = 32
EPS = 1e-5
SLOPE = 0.1

NCORE = 2
NSUB = 16
CH = 256
SB = 128
NSC = CH // SB
NBUF = 3
EP = 811008
NP = 50048
ROWS_PER_SUB = NP // NSUB


def _sc_spmm(table, src2, dst_r, w_p, split_cols):
  nchunks = (EP // NSUB) // CH if split_cols else (EP // (NCORE * NSUB)) // CH
  ntrip = nchunks // NBUF
  mesh = plsc.VectorSubcoreMesh(core_axis_name="c", subcore_axis_name="s")
  zeros = jnp.zeros((NP, F), jnp.float32)

  @functools.partial(
      pl.kernel,
      out_type=jax.ShapeDtypeStruct((NCORE, NP, F), jnp.float32),
      mesh=mesh,
      compiler_params=pltpu.CompilerParams(use_tc_tiling_on_sc=False),
      scratch_types=[
          pltpu.VMEM((NBUF, CH), jnp.int32),
          pltpu.VMEM((NBUF, NSC, SB), jnp.int32),
          pltpu.VMEM((NBUF, CH), jnp.float32),
          pltpu.VMEM((NBUF, CH, F), jnp.float32),
          pltpu.VMEM_SHARED((NP, F), jnp.float32),
          pltpu.SemaphoreType.DMA((NBUF,)),
          pltpu.SemaphoreType.DMA((NBUF,)),
          pltpu.SemaphoreType.DMA((NBUF,)),
      ],
  )
  def spmm(table_h, src_h, dstr_h, w_h, zeros_h, out_h,
           srcv, dstv, wv, rows, acc, gsem, ssem, lsem):
    c = lax.axis_index("c")
    s = lax.axis_index("s")

    r0 = pl.multiple_of(s * ROWS_PER_SUB, 8)
    pltpu.sync_copy(zeros_h.at[pl.ds(r0, ROWS_PER_SUB)],
                    acc.at[pl.ds(r0, ROWS_PER_SUB)])
    plsc.subcore_barrier()

    if split_cols:
      base = s * (EP // NSUB)
    else:
      base = (c * NSUB + s) * (EP // (NCORE * NSUB))

    def issue_linear_sw(k, b):
      e0 = pl.multiple_of(base + k * CH, CH)
      pltpu.async_copy(src_h.at[pl.ds(pl.multiple_of(c * EP + e0, CH), CH)],
                       srcv.at[b], lsem.at[b])
      pltpu.async_copy(w_h.at[pl.ds(e0, CH)], wv.at[b], lsem.at[b])

    def issue_linear_d(k, b):
      e0 = pl.multiple_of(base + k * CH, CH)
      pltpu.async_copy(dstr_h.at[pl.ds(pl.multiple_of(e0 // SB, NSC), NSC)],
                       dstv.at[b], lsem.at[b])

    def wait_linear(b):
      pltpu.make_async_copy(src_h.at[pl.ds(0, CH)], srcv.at[b],
                            lsem.at[b]).wait()
      pltpu.make_async_copy(w_h.at[pl.ds(0, CH)], wv.at[b],
                            lsem.at[b]).wait()
      pltpu.make_async_copy(dstr_h.at[pl.ds(0, NSC)], dstv.at[b],
                            lsem.at[b]).wait()

    def issue_gather(b):
      pltpu.async_copy(table_h.at[srcv.at[b]], rows.at[b], gsem.at[b])

    def wait_gather(b):
      pltpu.make_async_copy(zeros_h.at[pl.ds(0, CH)], rows.at[b],
                            gsem.at[b]).wait()

    def multiply(b):
      def group_body(g, _):
        w16 = wv[b, pl.ds(g * 16, 16)]
        for u in range(16):
          e = g * 16 + u
          ws = w16[u]
          for h in range(0, F, 16):
            rows[b, e, pl.ds(h, 16)] = rows[b, e, pl.ds(h, 16)] * ws
        return 0
      lax.fori_loop(0, CH // 16, group_body, 0)

    def issue_scatters(b):
      for j in range(NSC):
        pltpu.async_copy(rows.at[b, pl.ds(j * SB, SB)],
                         acc.at[dstv.at[b, j]], ssem.at[b], add=True)

    def drain_scatters(b):
      for j in range(NSC):
        pltpu.make_async_copy(zeros_h.at[pl.ds(0, SB)],
                              rows.at[b, pl.ds(j * SB, SB)],
                              ssem.at[b]).wait()

    for b in range(NBUF - 1):
      issue_linear_sw(b, b)
      issue_linear_d(b, b)
      wait_linear(b)
      issue_gather(b)

    def trip_body(t, _):
      for b in range(NBUF):
        k = t * NBUF + b
        bp = (b + NBUF - 1) % NBUF
        q = k + NBUF - 1

        if b == 0:
          issue_linear_sw(q, bp)
        else:
          @pl.when(t < ntrip - 1)
          def _():
            issue_linear_sw(q, bp)

        wait_gather(b)
        multiply(b)
        issue_scatters(b)

        def finish_refill():
          drain_scatters(bp)
          issue_linear_d(q, bp)
          wait_linear(bp)
          issue_gather(bp)

        if b == 0:
          @pl.when(t == 0)
          def _():
            issue_linear_d(q, bp)
            wait_linear(bp)
            issue_gather(bp)

          @pl.when(t > 0)
          def _():
            finish_refill()
        else:
          @pl.when(t < ntrip - 1)
          def _():
            finish_refill()

          @pl.when(t == ntrip - 1)
          def _():
            drain_scatters(bp)
      return 0

    lax.fori_loop(0, ntrip, trip_body, 0)
    drain_scatters((nchunks - 1) % NBUF)

    plsc.subcore_barrier()
    pltpu.sync_copy(acc.at[pl.ds(r0, ROWS_PER_SUB)],
                    out_h.at[c, pl.ds(r0, ROWS_PER_SUB)])

  return spmm(table, src2, dst_r, w_p, zeros)


BN = 2000


def _tc_mm_body(x_ref, s2_ref, a_ref, b2_ref, bias_ref, y_ref, s1_ref, sq_ref):
  y = jnp.dot(x_ref[...], a_ref[...], preferred_element_type=jnp.float32)
  y -= jnp.dot(s2_ref[0], b2_ref[0], preferred_element_type=jnp.float32)
  y -= jnp.dot(s2_ref[1], b2_ref[1], preferred_element_type=jnp.float32)
  y += bias_ref[...]
  y_ref[...] = y
  part = jnp.sum(y.reshape(BN // 8, 8, F), axis=0)
  psq = jnp.sum((y * y).reshape(BN // 8, 8, F), axis=0)

  @pl.when(pl.program_id(0) == 0)
  def _():
    s1_ref[...] = part
    sq_ref[...] = psq

  @pl.when(pl.program_id(0) != 0)
  def _():
    s1_ref[...] += part
    sq_ref[...] += psq


def _tc_matmul(x, s2, a, b2, bias):
  d = x.shape[1]
  return pl.pallas_call(
      _tc_mm_body,
      grid=(N // BN,),
      in_specs=[
          pl.BlockSpec((BN, d), lambda i: (i, 0)),
          pl.BlockSpec((2, BN, F), lambda i: (0, i, 0)),
          pl.BlockSpec((d, F), lambda i: (0, 0)),
          pl.BlockSpec((2, F, F), lambda i: (0, 0, 0)),
          pl.BlockSpec((1, F), lambda i: (0, 0)),
      ],
      out_specs=[
          pl.BlockSpec((BN, F), lambda i: (i, 0)),
          pl.BlockSpec((8, F), lambda i: (0, 0)),
          pl.BlockSpec((8, F), lambda i: (0, 0)),
      ],
      out_shape=[
          jax.ShapeDtypeStruct((N, F), jnp.float32),
          jax.ShapeDtypeStruct((8, F), jnp.float32),
          jax.ShapeDtypeStruct((8, F), jnp.float32),
      ],
  )(x, s2, a, b2, bias)


def _tc_norm_body(y_ref, s1_ref, sq_ref, g_ref, b_ref, o_ref):
  s1 = jnp.sum(s1_ref[...], axis=0, keepdims=True)
  sq = jnp.sum(sq_ref[...], axis=0, keepdims=True)
  mean = s1 / N
  var = sq / N - mean * mean
  scale = g_ref[...] * lax.rsqrt(var + EPS)
  shift = b_ref[...] - mean * scale
  o = y_ref[...] * scale + shift
  o_ref[...] = jnp.where(o >= 0, o, SLOPE * o)


def _tc_norm(y, s1, sq, gamma, beta):
  return pl.pallas_call(
      _tc_norm_body,
      grid=(N // BN,),
      in_specs=[
          pl.BlockSpec((BN, F), lambda i: (i, 0)),
          pl.BlockSpec((8, F), lambda i: (0, 0)),
          pl.BlockSpec((8, F), lambda i: (0, 0)),
          pl.BlockSpec((1, F), lambda i: (0, 0)),
          pl.BlockSpec((1, F), lambda i: (0, 0)),
      ],
      out_specs=pl.BlockSpec((BN, F), lambda i: (i, 0)),
      out_shape=jax.ShapeDtypeStruct((N, F), jnp.float32),
  )(y, s1, sq, gamma, beta)


def _combine_weights(Ws):
  ks = jnp.arange(Ws.shape[0], dtype=jnp.float32)
  return jnp.sum(Ws, axis=0), jnp.einsum("k,kij->ij", ks, Ws)


def _conv_block(x, table, src2, dst_r, w_p, Ws, bias, gamma, beta, split_cols):
  a, b = _combine_weights(Ws)
  if split_cols:
    b2 = jnp.stack([b[:F], b[F:]])
  else:
    b2 = jnp.stack([b, b])
  s2 = _sc_spmm(table, src2, dst_r, w_p, split_cols)
  y, s1, sq = _tc_matmul(x, s2[:, :N, :], a, b2, bias.reshape(1, F))
  return _tc_norm(y, s1, sq, gamma.reshape(1, F), beta.reshape(1, F))


def _prep_edges(ei, w):
  pad = EP - E
  src = jnp.concatenate([ei[0].astype(jnp.int32), jnp.zeros((pad,), jnp.int32)])
  dst = jnp.concatenate([ei[1].astype(jnp.int32), jnp.zeros((pad,), jnp.int32)])
  w_p = jnp.concatenate([w, jnp.zeros((pad,), jnp.float32)])
  src2_off = jnp.concatenate([src, src + N])
  src2_eq = jnp.concatenate([src, src])
  return src2_off, src2_eq, dst.reshape(EP // SB, SB), w_p


def _stream(x0, ei, w, W0, W1, bias, gamma, beta):
  src2_off, src2_eq, dst_r, w_p = _prep_edges(ei, w)
  table0 = jnp.concatenate([x0[:, :F], x0[:, F:]], axis=0)
  h = _conv_block(x0, table0, src2_off, dst_r, w_p,
                  W0, bias[0], gamma[0], beta[0], split_cols=True)
  return _conv_block(h, h, src2_eq, dst_r, w_p,
                     W1, bias[1], gamma[1], beta[1], split_cols=False)


def kernel(x_t0, edge_weight_t, x_s0, edge_weight_s, Wt0, Wt1, bias_t, gamma_t,
           beta_t, Ws0, Ws1, bias_s, gamma_s, beta_s, edge_index_t,
           edge_index_s):
  out_t = _stream(x_t0, edge_index_t, edge_weight_t,
                  Wt0, Wt1, bias_t, gamma_t, beta_t)
  out_s = _stream(x_s0, edge_index_s, edge_weight_s,
                  Ws0, Ws1, bias_s, gamma_s, beta_s)
  return (out_t, out_s)
